# baseline (speedup 1.0000x reference)
import functools

import jax
import jax.numpy as jnp
from jax.experimental import pallas as pl
from jax.experimental.pallas import tpu as pltpu

_LANE_PAD = 128
_HEAD = 2
_OUT = 4
_INV_TEMP = 1.0 / 5.0
_TANH_C = 2.5
_CHUNK = 128
_NGATES = 4


def _ctrl_kernel(idx_ref, h_ref, c_ref, emb_ref,
                 wi_ref, wf_ref, wg_ref, wo_ref,
                 bi_ref, bf_ref, bg_ref, bo_ref,
                 decw_ref, decb_ref,
                 logits_ref, hx_ref, cx_ref,
                 hbf_ref, acc_ref, *, hid, nsteps):
    n = pl.program_id(0)
    n_emb = emb_ref.shape[0]

    @pl.when(n == 0)
    def _():
        hbf_ref[...] = h_ref[...].astype(jnp.bfloat16)

    hbf = hbf_ref[...]
    embbf = emb_ref[...].astype(jnp.bfloat16)
    onehot = (idx_ref[...] == jax.lax.broadcasted_iota(
        jnp.int32, (1, n_emb), 1)).astype(jnp.bfloat16)

    gate_vals = []
    for w_ref, b_ref in ((wi_ref, bi_ref), (wf_ref, bf_ref),
                         (wg_ref, bg_ref), (wo_ref, bo_ref)):
        wbf = w_ref[...].astype(jnp.bfloat16)
        eg = jnp.dot(embbf, wbf[:hid, :],
                     preferred_element_type=jnp.float32)
        eg = (eg + b_ref[...]).astype(jnp.bfloat16)
        gx = jnp.dot(onehot, eg, preferred_element_type=jnp.float32)
        gh = jnp.dot(hbf, wbf[hid:, :],
                     preferred_element_type=jnp.float32)
        gate_vals.append(gx + gh)

    i_g = jax.nn.sigmoid(gate_vals[0])
    f_g = jax.nn.sigmoid(gate_vals[1])
    g_g = jnp.tanh(gate_vals[2])
    o_g = jax.nn.sigmoid(gate_vals[3])

    cx = f_g * c_ref[...] + i_g * g_g
    hx = o_g * jnp.tanh(cx)
    cx_ref[...] = cx
    hx_ref[...] = hx

    p = jnp.dot(hx.astype(jnp.bfloat16), decw_ref[...].astype(jnp.bfloat16),
                preferred_element_type=jnp.float32)

    @pl.when(n == 0)
    def _():
        acc_ref[...] = p

    @pl.when(n > 0)
    def _():
        acc_ref[...] += p

    @pl.when(n == nsteps - 1)
    def _():
        logits = acc_ref[...] + decb_ref[...]
        logits_ref[...] = (_TANH_C * jnp.tanh(logits * _INV_TEMP))[:, :_OUT]


@functools.partial(jax.jit, static_argnames=("batch", "hid"))
def _run(idx2, h0, c0, embedding, w_lstm, b_lstm, dec_w_pad, dec_b_pad,
         batch, hid):
    nsteps = hid // _CHUNK
    kernel_body = functools.partial(_ctrl_kernel, hid=hid, nsteps=nsteps)
    n_emb = embedding.shape[0]

    def wmap(k):
        return lambda n: (0, k * nsteps + n)

    def bmap(k):
        return lambda n: (0, k * nsteps + n)

    return pl.pallas_call(
        kernel_body,
        out_shape=(
            jax.ShapeDtypeStruct((batch, _OUT), jnp.float32),
            jax.ShapeDtypeStruct((batch, hid), jnp.float32),
            jax.ShapeDtypeStruct((batch, hid), jnp.float32),
        ),
        grid=(nsteps,),
        in_specs=[
            pl.BlockSpec((batch, 1), lambda n: (0, 0)),
            pl.BlockSpec((batch, hid), lambda n: (0, 0)),
            pl.BlockSpec((batch, _CHUNK), lambda n: (0, n)),
            pl.BlockSpec((n_emb, hid), lambda n: (0, 0)),
            pl.BlockSpec((2 * hid, _CHUNK), wmap(0)),
            pl.BlockSpec((2 * hid, _CHUNK), wmap(1)),
            pl.BlockSpec((2 * hid, _CHUNK), wmap(2)),
            pl.BlockSpec((2 * hid, _CHUNK), wmap(3)),
            pl.BlockSpec((1, _CHUNK), bmap(0)),
            pl.BlockSpec((1, _CHUNK), bmap(1)),
            pl.BlockSpec((1, _CHUNK), bmap(2)),
            pl.BlockSpec((1, _CHUNK), bmap(3)),
            pl.BlockSpec((None, _CHUNK, _LANE_PAD),
                         lambda n: (_HEAD, n, 0)),
            pl.BlockSpec((None, 1, _LANE_PAD),
                         lambda n: (_HEAD, 0, 0)),
        ],
        out_specs=(
            pl.BlockSpec((batch, _OUT), lambda n: (0, 0)),
            pl.BlockSpec((batch, _CHUNK), lambda n: (0, n)),
            pl.BlockSpec((batch, _CHUNK), lambda n: (0, n)),
        ),
        scratch_shapes=[
            pltpu.VMEM((batch, hid), jnp.bfloat16),
            pltpu.VMEM((batch, _LANE_PAD), jnp.float32),
        ],
        compiler_params=pltpu.CompilerParams(
            dimension_semantics=("arbitrary",)),
    )(idx2, h0, c0, embedding,
      w_lstm, w_lstm, w_lstm, w_lstm,
      b_lstm, b_lstm, b_lstm, b_lstm,
      dec_w_pad, dec_b_pad)


def kernel(inputs, h0, c0, embedding, w_lstm, b_lstm, dec_w_pad, dec_b_pad):
    batch = inputs.shape[0]
    hid = h0.shape[1]
    idx2 = inputs.reshape(batch, 1)
    logits, hx, cx = _run(idx2, h0, c0, embedding, w_lstm, b_lstm,
                          dec_w_pad, dec_b_pad, batch=batch, hid=hid)
    return logits, (hx, cx)
